# baseline (speedup 1.0000x reference)
import functools

import jax
import jax.numpy as jnp
from jax import lax
from jax.experimental import pallas as pl
from jax.experimental.pallas import tpu as pltpu
from jax.experimental.pallas import tpu_sc as plsc

_D = 64
_NWORDS = 430080
_NBAGS = 21504
_WORKERS = 32
_BPW = _NBAGS // _WORKERS
_NPASS = 6
_HPW = _BPW // _NPASS
_NTRASH = 8
_REG = _HPW + _NTRASH
_FT = 56
_C = 512
_CB = 128
_NS = 16
_LOFF = _BPW + 24


def _sload(ref, i):
    return ref[pl.ds(i, 16)][0]


def _search_last_le(loff, limit, lo0, hi0):
    lo, hi = lo0, hi0
    for _ in range(10):
        mid = (lo + hi + 1) // 2
        take = _sload(loff, mid) <= limit
        lo = jnp.where(take, mid, lo)
        hi = jnp.where(take, hi, mid - 1)
    return lo


def _sc_bag_means(words_pad, offsets_pad, table128):
    mesh = plsc.VectorSubcoreMesh(core_axis_name="c", subcore_axis_name="s")

    @functools.partial(
        pl.kernel,
        out_type=jax.ShapeDtypeStruct((_NBAGS, _D), jnp.float32),
        mesh=mesh,
        scratch_types=[
            pltpu.VMEM((_LOFF,), jnp.int32),
            pltpu.VMEM((_C // _CB, _CB), jnp.int32),
            pltpu.VMEM((_C // _CB, _CB), jnp.int32),
            pltpu.VMEM((_C // _CB, _CB), jnp.int32),
            pltpu.VMEM((_C // _CB, _CB), jnp.int32),
            pltpu.VMEM((_C // _CB, _CB), jnp.int32),
            pltpu.VMEM((_C, 2 * _D), jnp.float32),
            pltpu.VMEM((_FT, 2 * _D), jnp.float32),
            pltpu.VMEM((_FT, 2 * _D), jnp.float32),
            pltpu.VMEM((_FT, _D), jnp.float32),
            pltpu.VMEM_SHARED((_NS * 2 * _REG, 2 * _D), jnp.float32),
            pltpu.SemaphoreType.DMA,
            pltpu.SemaphoreType.DMA,
            pltpu.SemaphoreType.DMA,
        ],
        compiler_params=pltpu.CompilerParams(use_tc_tiling_on_sc=True),
    )
    def k(words_ref, offs_ref, table_ref, out_ref,
          loff, widx, gidx, bidx, eidx, oidx, rows, fbe, fbo, fbm,
          acc, sem_g, sem_s, sem_w):
        c = lax.axis_index("c")
        s = lax.axis_index("s")
        wid = c * _NS + s
        bag0 = wid * _BPW
        reg_e = s * (2 * _REG)
        reg_o = reg_e + _REG

        pltpu.sync_copy(offs_ref.at[pl.ds(bag0, _LOFF)], loff)

        iota = lax.iota(jnp.int32, 16)

        for h in range(_NPASS):
            hb = h * _HPW

            def zb(r, _):
                for kk in range(2 * _D // 16):
                    fbe[r, pl.ds(kk * 16, 16)] = jnp.zeros((16,), jnp.float32)
                return 0
            lax.fori_loop(0, _FT, zb, 0)
            for base in (reg_e, reg_o):
                for t in range(_REG // _FT):
                    pltpu.sync_copy(fbe, acc.at[pl.ds(base + t * _FT, _FT)])
                pltpu.sync_copy(fbe.at[pl.ds(0, _REG % _FT)],
                                acc.at[pl.ds(base + (_REG // _FT) * _FT,
                                             _REG % _FT)])

            trash_e = (jnp.zeros((16,), jnp.int32) + (reg_e + _HPW)
                       + (iota & (_NTRASH - 1)))
            trash_o = (jnp.zeros((16,), jnp.int32) + (reg_o + _HPW)
                       + (iota & (_NTRASH - 1)))

            w_start = _sload(loff, hb)
            w_end = _sload(loff, hb + _HPW)
            cs0 = (w_start // 8) * 8
            n_chunks = (w_end - cs0 + _C - 1) // _C

            @pl.loop(0, n_chunks)
            def _chunk(ci):
                cs = cs0 + ci * _C
                csa = pl.multiple_of(cs, 8)
                pos_last = cs + _C - 1

                wcps = [pltpu.async_copy(
                            words_ref.at[pl.ds(csa + j * _CB, _CB)],
                            widx.at[j], sem_w)
                        for j in range(_C // _CB)]

                sent = jnp.zeros((16,), jnp.int32) + _HPW
                for g in range(_C // 16):
                    bidx[g // (_CB // 16),
                         pl.ds((g % (_CB // 16)) * 16, 16)] = sent
                b_lo = _search_last_le(loff, cs, jnp.int32(hb),
                                       jnp.int32(hb + _HPW))
                b_hi = _search_last_le(loff, pos_last, b_lo,
                                       jnp.int32(hb + _HPW))

                @pl.loop(b_lo, b_hi + 1)
                def _bag(b):
                    s0 = jnp.maximum(_sload(loff, b) - cs, 0)
                    e0 = jnp.minimum(_sload(loff, b + 1) - cs, _C)
                    sv = (jnp.zeros((16,), jnp.int32)
                          + jnp.minimum(b - hb, _HPW))

                    @pl.loop(s0 // 16, (e0 + 15) // 16)
                    def _grp(g):
                        gp = g * 16 + iota
                        mask = jnp.logical_and(gp >= s0, gp < e0)
                        row = g // (_CB // 16)
                        col = (g % (_CB // 16)) * 16
                        cur = bidx[row, pl.ds(col, 16)]
                        bidx[row, pl.ds(col, 16)] = jnp.where(mask, sv, cur)

                for cp in wcps:
                    cp.wait()

                for g in range(_C // 16):
                    row = g // (_CB // 16)
                    col = (g % (_CB // 16)) * 16
                    wv = widx[row, pl.ds(col, 16)]
                    bv = bidx[row, pl.ds(col, 16)]
                    par = wv & 1
                    ok_e = jnp.logical_and(par == 0, bv < _HPW)
                    ok_o = jnp.logical_and(par == 1, bv < _HPW)
                    gidx[row, pl.ds(col, 16)] = wv >> 1
                    eidx[row, pl.ds(col, 16)] = \
                        jnp.where(ok_e, reg_e + bv, trash_e)
                    oidx[row, pl.ds(col, 16)] = \
                        jnp.where(ok_o, reg_o + bv, trash_o)

                gcps = [pltpu.async_copy(table_ref.at[gidx.at[j]],
                                         rows.at[pl.ds(j * _CB, _CB)], sem_g)
                        for j in range(_C // _CB)]
                for cp in gcps:
                    cp.wait()
                scps = []
                for j in range(_C // _CB):
                    scps.append(pltpu.async_copy(
                        rows.at[pl.ds(j * _CB, _CB)],
                        acc.at[eidx.at[j]], sem_s, add=True))
                    scps.append(pltpu.async_copy(
                        rows.at[pl.ds(j * _CB, _CB)],
                        acc.at[oidx.at[j]], sem_s, add=True))
                for cp in scps:
                    cp.wait()

            def fin_t(t, _):
                pltpu.async_copy(acc.at[pl.ds(reg_e + t * _FT, _FT)], fbe,
                                 sem_w).wait()
                pltpu.async_copy(acc.at[pl.ds(reg_o + t * _FT, _FT)], fbo,
                                 sem_w).wait()

                def fin_b(b, _):
                    i = hb + t * _FT + b
                    ov = loff[pl.ds(i, 16)]
                    cnt = ov[1] - ov[0]
                    den = jnp.maximum(
                        (jnp.zeros((16,), jnp.int32) + cnt)
                        .astype(jnp.float32), 1.0)
                    for kk in range(_D // 16):
                        sm = (fbe[b, pl.ds(kk * 16, 16)]
                              + fbo[b, pl.ds(_D + kk * 16, 16)])
                        fbm[b, pl.ds(kk * 16, 16)] = sm / den
                    return 0

                lax.fori_loop(0, _FT, fin_b, 0)
                pltpu.async_copy(
                    fbm, out_ref.at[pl.ds(bag0 + hb + t * _FT, _FT)],
                    sem_w).wait()
                return 0

            lax.fori_loop(0, _HPW // _FT, fin_t, 0)

    return k(words_pad, offsets_pad, table128)


def _tc_loss(means):
    x = means.reshape(_NBAGS // 21, 21, _D)

    def body(x_ref, o_ref):
        xx = x_ref[...]
        src = xx[:, 0, :]
        tgt = xx[:, 1:, :]
        scores = jnp.sum(tgt * src[:, None, :], axis=-1)
        m = jnp.max(scores, axis=1)
        lse = jnp.log(jnp.sum(jnp.exp(scores - m[:, None]), axis=1)) + m
        o_ref[...] = jnp.mean(lse - scores[:, 0]).reshape(1, 1)

    out = pl.pallas_call(
        body, out_shape=jax.ShapeDtypeStruct((1, 1), jnp.float32))(x)
    return out[0, 0]


def kernel(words, offsets, emb_table):
    words = words.astype(jnp.int32)
    offsets = offsets.astype(jnp.int32)
    words_pad = jnp.concatenate(
        [words, jnp.zeros((_C + 8,), jnp.int32)])
    offsets_pad = jnp.concatenate(
        [offsets, jnp.full((24,), _NWORDS, jnp.int32)])
    table128 = emb_table.astype(jnp.float32).reshape(500000, 2 * _D)
    means = _sc_bag_means(words_pad, offsets_pad, table128)
    return _tc_loss(means)


# speedup vs baseline: 1.3047x; 1.3047x over previous
import functools

import jax
import jax.numpy as jnp
from jax import lax
from jax.experimental import pallas as pl
from jax.experimental.pallas import tpu as pltpu
from jax.experimental.pallas import tpu_sc as plsc

_D = 64
_NWORDS = 430080
_NBAGS = 21504
_WORKERS = 32
_BPW = _NBAGS // _WORKERS
_ACC_ROWS = _BPW + 1
_C = 512
_CB = 128
_NS = 16
_LOFF = _BPW + 24


def _sload(ref, i):
    return ref[pl.ds(i, 16)][0]


def _search_last_le(loff, limit, lo0):
    lo, hi = lo0, jnp.int32(_BPW)
    for _ in range(10):
        mid = (lo + hi + 1) // 2
        take = _sload(loff, mid) <= limit
        lo = jnp.where(take, mid, lo)
        hi = jnp.where(take, hi, mid - 1)
    return lo


def _sc_bag_means(words_pad, offsets_pad, table, zeros_rows):
    mesh = plsc.VectorSubcoreMesh(core_axis_name="c", subcore_axis_name="s")

    @functools.partial(
        pl.kernel,
        out_type=jax.ShapeDtypeStruct((_NBAGS, _D), jnp.float32),
        mesh=mesh,
        scratch_types=[
            pltpu.VMEM((_LOFF,), jnp.int32),
            pltpu.VMEM((_C // _CB, _CB), jnp.int32),
            pltpu.VMEM((_C // _CB, _CB), jnp.int32),
            pltpu.VMEM((_C, _D), jnp.float32),
            pltpu.VMEM((96, _D), jnp.float32),
            pltpu.VMEM_SHARED((_NS * _ACC_ROWS, _D), jnp.float32),
            pltpu.SemaphoreType.DMA((_C // _CB,)),
            pltpu.SemaphoreType.DMA,
            pltpu.SemaphoreType.DMA,
        ],
        compiler_params=pltpu.CompilerParams(use_tc_tiling_on_sc=False),
    )
    def k(words_ref, offs_ref, table_ref, zrows_ref, out_ref,
          loff, widx, sidx, rows, fbuf, acc, sem_g, sem_s, sem_w):
        c = lax.axis_index("c")
        s = lax.axis_index("s")
        wid = c * _NS + s
        bag0 = wid * _BPW
        abase = s * _ACC_ROWS

        pltpu.sync_copy(offs_ref.at[pl.ds(bag0, _LOFF)], loff)
        pltpu.sync_copy(zrows_ref, acc.at[pl.ds(abase, _ACC_ROWS)])

        w_start = _sload(loff, 0)
        w_end = _sload(loff, _BPW)
        cs0 = (w_start // _C) * _C
        n_chunks = (w_end - cs0 + _C - 1) // _C
        iota = lax.iota(jnp.int32, 16)
        trash_v = jnp.zeros((16,), jnp.int32) + (abase + _BPW)

        @pl.loop(0, n_chunks)
        def _chunk(ci):
            cs = cs0 + ci * _C
            csa = pl.multiple_of(cs, _C)
            pos_last = cs + _C - 1

            pltpu.async_copy(words_ref.at[pl.ds(csa // _CB, _C // _CB)],
                             widx, sem_w).wait()
            gcps = [pltpu.async_copy(table_ref.at[widx.at[j]],
                                     rows.at[pl.ds(j * _CB, _CB)],
                                     sem_g.at[j])
                    for j in range(_C // _CB)]

            for g in range(_C // 16):
                sidx[g // (_CB // 16), pl.ds((g % (_CB // 16)) * 16, 16)] = \
                    trash_v
            b_lo = _search_last_le(loff, cs, jnp.int32(0))
            b_hi = _search_last_le(loff, pos_last, b_lo)

            @pl.loop(b_lo, b_hi + 1)
            def _bag(b):
                s0 = jnp.maximum(_sload(loff, b) - cs, 0)
                e0 = jnp.minimum(_sload(loff, b + 1) - cs, _C)
                sv = jnp.zeros((16,), jnp.int32) + (abase + b)

                @pl.loop(s0 // 16, (e0 + 15) // 16)
                def _grp(g):
                    gp = g * 16 + iota
                    mask = jnp.logical_and(gp >= s0, gp < e0)
                    row = g // (_CB // 16)
                    col = (g % (_CB // 16)) * 16
                    cur = sidx[row, pl.ds(col, 16)]
                    sidx[row, pl.ds(col, 16)] = jnp.where(mask, sv, cur)

            scps = []
            for j in range(_C // _CB):
                gcps[j].wait()
                scps.append(pltpu.async_copy(rows.at[pl.ds(j * _CB, _CB)],
                                             acc.at[sidx.at[j]], sem_s,
                                             add=True))
            for cp in scps:
                cp.wait()

        def fin_t(t, _):
            pltpu.async_copy(acc.at[pl.ds(abase + t * 96, 96)], fbuf,
                             sem_w).wait()

            def fin_b(b, _):
                i = t * 96 + b
                ov = loff[pl.ds(i, 16)]
                cnt = ov[1] - ov[0]
                den = jnp.maximum(
                    (jnp.zeros((16,), jnp.int32) + cnt).astype(jnp.float32),
                    1.0)
                for kk in range(_D // 16):
                    fbuf[b, pl.ds(kk * 16, 16)] = (
                        fbuf[b, pl.ds(kk * 16, 16)] / den)
                return 0

            lax.fori_loop(0, 96, fin_b, 0)
            pltpu.async_copy(fbuf, out_ref.at[pl.ds(bag0 + t * 96, 96)],
                             sem_w).wait()
            return 0

        lax.fori_loop(0, _BPW // 96, fin_t, 0)

    return k(words_pad, offsets_pad, table, zeros_rows)


def _tc_loss(means):
    x = means.reshape(_NBAGS // 21, 21, _D)

    def body(x_ref, o_ref):
        xx = x_ref[...]
        src = xx[:, 0, :]
        tgt = xx[:, 1:, :]
        scores = jnp.sum(tgt * src[:, None, :], axis=-1)
        m = jnp.max(scores, axis=1)
        lse = jnp.log(jnp.sum(jnp.exp(scores - m[:, None]), axis=1)) + m
        o_ref[...] = jnp.mean(lse - scores[:, 0]).reshape(1, 1)

    out = pl.pallas_call(
        body, out_shape=jax.ShapeDtypeStruct((1, 1), jnp.float32))(x)
    return out[0, 0]


def kernel(words, offsets, emb_table):
    words = words.astype(jnp.int32)
    offsets = offsets.astype(jnp.int32)
    words_pad = jnp.concatenate(
        [words, jnp.zeros((_C + _CB,), jnp.int32)]).reshape(-1, _CB)
    offsets_pad = jnp.concatenate(
        [offsets, jnp.full((24,), _NWORDS, jnp.int32)])
    zeros_rows = jnp.zeros((_ACC_ROWS, _D), jnp.float32)
    means = _sc_bag_means(words_pad, offsets_pad,
                          emb_table.astype(jnp.float32), zeros_rows)
    return _tc_loss(means)
